# baseline (speedup 1.0000x reference)
import functools

import jax
import jax.numpy as jnp
from jax import lax
from jax.experimental import pallas as pl
from jax.experimental.pallas import tpu as pltpu
from jax.experimental.pallas import tpu_sc as plsc

N_NODES = 4096
N_EDGES = 262144
F_EDGE = 16

NC = 2
NS = 16
NW = NC * NS
EPW = N_EDGES // NW
CE = 1024
NCH = EPW // CE
GP = CE // 16


def _sc_scatter_body(idx_hbm, eat_hbm, zt_hbm, out_hbm,
                     iv0, iv1, et0, et1, acct, iotav, acc_sh,
                     sem_i0, sem_i1, sem_e0, sem_e1):
    c = lax.axis_index("c")
    s = lax.axis_index("s")
    w = c * NS + s

    bufs = ((iv0, et0, sem_i0, sem_e0), (iv1, et1, sem_i1, sem_e1))

    def start(k):
        iv, et, si, se = bufs[k % 2]
        base_e = pl.multiple_of(w * EPW + k * CE, CE)
        return (pltpu.async_copy(idx_hbm.at[0, pl.ds(base_e, CE)], iv, si),
                pltpu.async_copy(eat_hbm.at[:, pl.ds(base_e, CE)], et, se))

    pend = start(0)

    pltpu.sync_copy(zt_hbm, acct)
    r0 = pl.multiple_of(s * (N_NODES // NS), N_NODES // NS)
    pltpu.sync_copy(zt_hbm.at[:, pl.ds(r0, N_NODES // NS)],
                    acc_sh.at[:, pl.ds(r0, N_NODES // NS)])
    iotav[...] = lax.iota(jnp.int32, 16)

    fvecs = [jnp.full((16,), f, jnp.int32) for f in range(F_EDGE)]

    for k in range(NCH):
        nxt = start(k + 1) if k + 1 < NCH else None
        for d in pend:
            d.wait()
        iv, et, _, _ = bufs[k % 2]

        @plsc.parallel_loop(0, GP, 1, unroll=4)
        def g_body(g, iv=iv, et=et):
            o = pl.multiple_of(g * 16, 16)
            idxv = iv[pl.ds(o, 16)]
            for f in range(F_EDGE):
                vals = et[f, pl.ds(o, 16)]
                plsc.addupdate_scatter(acct.at[f], [idxv], vals)
        pend = nxt

    plsc.subcore_barrier()
    pltpu.sync_copy(acct, acc_sh.at[iotav], add=True)
    plsc.subcore_barrier()
    pltpu.sync_copy(acc_sh.at[:, pl.ds(r0, N_NODES // NS)],
                    out_hbm.at[c, :, pl.ds(r0, N_NODES // NS)])


@functools.partial(
    pl.kernel,
    out_type=jax.ShapeDtypeStruct((NC, F_EDGE, N_NODES), jnp.float32),
    mesh=plsc.VectorSubcoreMesh(core_axis_name="c", subcore_axis_name="s"),
    scratch_types=[
        pltpu.VMEM((CE,), jnp.int32),
        pltpu.VMEM((CE,), jnp.int32),
        pltpu.VMEM((F_EDGE, CE), jnp.float32),
        pltpu.VMEM((F_EDGE, CE), jnp.float32),
        pltpu.VMEM((F_EDGE, N_NODES), jnp.float32),
        pltpu.VMEM((16,), jnp.int32),
        pltpu.VMEM_SHARED((F_EDGE, N_NODES), jnp.float32),
        pltpu.SemaphoreType.DMA,
        pltpu.SemaphoreType.DMA,
        pltpu.SemaphoreType.DMA,
        pltpu.SemaphoreType.DMA,
    ],
    compiler_params=pltpu.CompilerParams(use_tc_tiling_on_sc=False,
                                         needs_layout_passes=False),
)
def _sc_scatter(idx_hbm, eat_hbm, zt_hbm, out_hbm, *rest):
    _sc_scatter_body(idx_hbm, eat_hbm, zt_hbm, out_hbm, *rest)


def _mlp_body(nf_ref, agg32_ref, w1a_ref, w1b_ref, b1_ref, w2_ref, b2_ref,
              we_ref, be_ref, emb_ref, wt_ref, sur_ref, suc_ref):
    agg_t = agg32_ref[0] + agg32_ref[1]
    x = nf_ref[...]
    h_agg = lax.dot_general(agg_t, w1b_ref[...], (((0,), (0,)), ((), ())),
                            preferred_element_type=jnp.float32)
    h = (jnp.dot(x, w1a_ref[...], preferred_element_type=jnp.float32)
         + h_agg + b1_ref[...])
    h = jnp.dot(h, w2_ref[...], preferred_element_type=jnp.float32) + b2_ref[...]
    emb = jnp.dot(h, we_ref[...], preferred_element_type=jnp.float32) + be_ref[...]
    emb_ref[...] = emb
    wt_ref[...] = (-10.0 * emb).T
    su = 5.0 * jnp.sum(emb * emb, axis=1, keepdims=True)
    sur_ref[...] = su
    suc_ref[...] = su.T - 0.5


def _mlp(node_feats, agg32, w1a, w1b, b1, w2, b2, we, be):
    return pl.pallas_call(
        _mlp_body,
        out_shape=(
            jax.ShapeDtypeStruct((N_NODES, 2), jnp.float32),
            jax.ShapeDtypeStruct((2, N_NODES), jnp.float32),
            jax.ShapeDtypeStruct((N_NODES, 1), jnp.float32),
            jax.ShapeDtypeStruct((1, N_NODES), jnp.float32),
        ),
    )(node_feats, agg32, w1a, w1b, b1, w2, b2, we, be)


DEC_B = 1024


def _decode_body(emb_ref, wt_ref, sur_ref, suc_ref, out_ref):
    pid = pl.program_id(0)
    m = (emb_ref[:, 0:1] * wt_ref[0:1, :] + emb_ref[:, 1:2] * wt_ref[1:2, :])
    t = m + sur_ref[...] + suc_ref[...]
    val = 0.5 + 0.5 * jnp.tanh(t)
    out_ref[...] = val
    c0 = pl.multiple_of(pid * DEC_B, DEC_B)
    sub = out_ref[:, pl.ds(c0, DEC_B)]
    eq = (lax.broadcasted_iota(jnp.int32, (DEC_B, DEC_B), 0)
          == lax.broadcasted_iota(jnp.int32, (DEC_B, DEC_B), 1))
    out_ref[:, pl.ds(c0, DEC_B)] = jnp.where(eq, 0.0, sub)


def _decode(emb, wt, sur, suc):
    nb = N_NODES // DEC_B
    return pl.pallas_call(
        _decode_body,
        grid=(nb,),
        in_specs=[
            pl.BlockSpec((DEC_B, 2), lambda i: (i, 0)),
            pl.BlockSpec((2, N_NODES), lambda i: (0, 0)),
            pl.BlockSpec((DEC_B, 1), lambda i: (i, 0)),
            pl.BlockSpec((1, N_NODES), lambda i: (0, 0)),
        ],
        out_specs=pl.BlockSpec((DEC_B, N_NODES), lambda i: (i, 0)),
        out_shape=jax.ShapeDtypeStruct((N_NODES, N_NODES), jnp.float32),
    )(emb, wt, sur, suc)


def kernel(node_feats, edge_index, edge_attr, W1, b1, W2, b2, W_emb, b_emb):
    zt = jnp.zeros((F_EDGE, N_NODES), jnp.float32)
    agg32 = _sc_scatter(edge_index.astype(jnp.int32), edge_attr.T, zt)
    emb, wt, sur, suc = _mlp(
        node_feats, agg32,
        W1[:128, :], W1[128:, :], b1[None, :],
        W2, b2[None, :], W_emb, b_emb[None, :],
    )
    adj = _decode(emb, wt, sur, suc)
    return (adj, emb)


# speedup vs baseline: 1.0115x; 1.0115x over previous
import functools

import jax
import jax.numpy as jnp
from jax import lax
from jax.experimental import pallas as pl
from jax.experimental.pallas import tpu as pltpu
from jax.experimental.pallas import tpu_sc as plsc

N_NODES = 4096
N_EDGES = 262144
F_EDGE = 16

NC = 2
NS = 16
NW = NC * NS
EPW = N_EDGES // NW
CE = 1024
NCH = EPW // CE
GP = CE // 16


def _sc_scatter_body(idx_hbm, eat_hbm, zt_hbm, out_hbm,
                     iv0, iv1, et0, et1, acct, iotav, acc_sh,
                     sem_i0, sem_i1, sem_e0, sem_e1):
    c = lax.axis_index("c")
    s = lax.axis_index("s")
    w = c * NS + s

    bufs = ((iv0, et0, sem_i0, sem_e0), (iv1, et1, sem_i1, sem_e1))

    def start(k):
        iv, et, si, se = bufs[k % 2]
        base_e = pl.multiple_of(w * EPW + k * CE, CE)
        return (pltpu.async_copy(idx_hbm.at[0, pl.ds(base_e, CE)], iv, si),
                pltpu.async_copy(eat_hbm.at[:, pl.ds(base_e, CE)], et, se))

    pend = start(0)

    pltpu.sync_copy(zt_hbm, acct)
    r0 = pl.multiple_of(s * (N_NODES // NS), N_NODES // NS)
    pltpu.sync_copy(zt_hbm.at[:, pl.ds(r0, N_NODES // NS)],
                    acc_sh.at[:, pl.ds(r0, N_NODES // NS)])
    iotav[...] = lax.iota(jnp.int32, 16)

    fvecs = [jnp.full((16,), f, jnp.int32) for f in range(F_EDGE)]

    for k in range(NCH):
        nxt = start(k + 1) if k + 1 < NCH else None
        for d in pend:
            d.wait()
        iv, et, _, _ = bufs[k % 2]

        @plsc.parallel_loop(0, GP, 1, unroll=4)
        def g_body(g, iv=iv, et=et):
            o = pl.multiple_of(g * 16, 16)
            idxv = iv[pl.ds(o, 16)]
            for f in range(F_EDGE):
                vals = et[f, pl.ds(o, 16)]
                plsc.addupdate_scatter(acct.at[f], [idxv], vals)
        pend = nxt

    plsc.subcore_barrier()
    pltpu.sync_copy(acct, acc_sh.at[iotav], add=True)
    plsc.subcore_barrier()
    pltpu.sync_copy(acc_sh.at[:, pl.ds(r0, N_NODES // NS)],
                    out_hbm.at[c, :, pl.ds(r0, N_NODES // NS)])


@functools.partial(
    pl.kernel,
    out_type=jax.ShapeDtypeStruct((NC, F_EDGE, N_NODES), jnp.float32),
    mesh=plsc.VectorSubcoreMesh(core_axis_name="c", subcore_axis_name="s"),
    scratch_types=[
        pltpu.VMEM((CE,), jnp.int32),
        pltpu.VMEM((CE,), jnp.int32),
        pltpu.VMEM((F_EDGE, CE), jnp.float32),
        pltpu.VMEM((F_EDGE, CE), jnp.float32),
        pltpu.VMEM((F_EDGE, N_NODES), jnp.float32),
        pltpu.VMEM((16,), jnp.int32),
        pltpu.VMEM_SHARED((F_EDGE, N_NODES), jnp.float32),
        pltpu.SemaphoreType.DMA,
        pltpu.SemaphoreType.DMA,
        pltpu.SemaphoreType.DMA,
        pltpu.SemaphoreType.DMA,
    ],
    compiler_params=pltpu.CompilerParams(use_tc_tiling_on_sc=False,
                                         needs_layout_passes=False),
)
def _sc_scatter(idx_hbm, eat_hbm, zt_hbm, out_hbm, *rest):
    _sc_scatter_body(idx_hbm, eat_hbm, zt_hbm, out_hbm, *rest)


def _mlp_body(nf_ref, agg32_ref, w1a_ref, w1b_ref, b1_ref, w2_ref, b2_ref,
              we_ref, be_ref, emb_ref, wt_ref, sur_ref, suc_ref):
    agg_t = agg32_ref[0] + agg32_ref[1]
    x = nf_ref[...]
    h_agg = lax.dot_general(agg_t, w1b_ref[...], (((0,), (0,)), ((), ())),
                            preferred_element_type=jnp.float32)
    h = (jnp.dot(x, w1a_ref[...], preferred_element_type=jnp.float32)
         + h_agg + b1_ref[...])
    h = jnp.dot(h, w2_ref[...], preferred_element_type=jnp.float32) + b2_ref[...]
    emb = jnp.dot(h, we_ref[...], preferred_element_type=jnp.float32) + be_ref[...]
    emb_ref[...] = emb
    wt_ref[...] = (-10.0 * emb).T
    su = 5.0 * jnp.sum(emb * emb, axis=1, keepdims=True)
    sur_ref[...] = su
    suc_ref[...] = su.T - 0.5


def _mlp(node_feats, agg32, w1a, w1b, b1, w2, b2, we, be):
    return pl.pallas_call(
        _mlp_body,
        out_shape=(
            jax.ShapeDtypeStruct((N_NODES, 2), jnp.float32),
            jax.ShapeDtypeStruct((2, N_NODES), jnp.float32),
            jax.ShapeDtypeStruct((N_NODES, 1), jnp.float32),
            jax.ShapeDtypeStruct((1, N_NODES), jnp.float32),
        ),
    )(node_feats, agg32, w1a, w1b, b1, w2, b2, we, be)


DEC_B = 512


def _decode_body(emb_ref, wt_ref, sur_ref, suc_ref, out_ref):
    pid = pl.program_id(0)
    m = (emb_ref[:, 0:1] * wt_ref[0:1, :] + emb_ref[:, 1:2] * wt_ref[1:2, :])
    t = m + sur_ref[...] + suc_ref[...]
    val = 0.5 + 0.5 * jnp.tanh(t)
    out_ref[...] = val
    c0 = pl.multiple_of(pid * DEC_B, DEC_B)
    sub = out_ref[:, pl.ds(c0, DEC_B)]
    eq = (lax.broadcasted_iota(jnp.int32, (DEC_B, DEC_B), 0)
          == lax.broadcasted_iota(jnp.int32, (DEC_B, DEC_B), 1))
    out_ref[:, pl.ds(c0, DEC_B)] = jnp.where(eq, 0.0, sub)


def _decode(emb, wt, sur, suc):
    nb = N_NODES // DEC_B
    return pl.pallas_call(
        _decode_body,
        grid=(nb,),
        in_specs=[
            pl.BlockSpec((DEC_B, 2), lambda i: (i, 0)),
            pl.BlockSpec((2, N_NODES), lambda i: (0, 0)),
            pl.BlockSpec((DEC_B, 1), lambda i: (i, 0)),
            pl.BlockSpec((1, N_NODES), lambda i: (0, 0)),
        ],
        out_specs=pl.BlockSpec((DEC_B, N_NODES), lambda i: (i, 0)),
        out_shape=jax.ShapeDtypeStruct((N_NODES, N_NODES), jnp.float32),
    )(emb, wt, sur, suc)


def kernel(node_feats, edge_index, edge_attr, W1, b1, W2, b2, W_emb, b_emb):
    zt = jnp.zeros((F_EDGE, N_NODES), jnp.float32)
    agg32 = _sc_scatter(edge_index.astype(jnp.int32), edge_attr.T, zt)
    emb, wt, sur, suc = _mlp(
        node_feats, agg32,
        W1[:128, :], W1[128:, :], b1[None, :],
        W2, b2[None, :], W_emb, b_emb[None, :],
    )
    adj = _decode(emb, wt, sur, suc)
    return (adj, emb)
